# baseline (speedup 1.0000x reference)
import functools

import jax
import jax.numpy as jnp
from jax import lax
from jax.experimental import pallas as pl
from jax.experimental.pallas import tpu as pltpu
from jax.experimental.pallas import tpu_sc as plsc

N = 10000
E = 320000
NFEAT = 128
NHID = 8
NHEADS = 8
NCLASS = 40

NTILE = 32
EPT = E // NTILE
C = 80
NCH = EPT // C
ROWS1 = 72
ROWS2 = 48
NP = 10240
RPT = NP // 16

_BN = 1000


def _tc1_body(x_ref, w1_ref, as_ref, ad_ref, t1_ref, a1_ref):
    h = jnp.dot(x_ref[...], w1_ref[...], preferred_element_type=jnp.float32)
    als = jnp.dot(h, as_ref[...], preferred_element_type=jnp.float32)
    ald = jnp.dot(h, ad_ref[...], preferred_element_type=jnp.float32)
    t1_ref[...] = jnp.concatenate([h, als], axis=1)
    a1_ref[...] = ald


def _tc2_body(acc_ref, b1_ref, w2_ref, as2_ref, ad2_ref, e8_ref, t2_ref, a2_ref):
    a0 = acc_ref[0]
    a1 = acc_ref[1]
    num = a0[:, :64] + a1[:, :64]
    den = a0[:, 64:] + a1[:, 64:]
    r = 1.0 / (den + 1e-16)
    rexp = jnp.dot(r, e8_ref[...], preferred_element_type=jnp.float32)
    hid = num * rexp + b1_ref[...]
    hid = jnp.where(hid > 0, hid, jnp.exp(hid) - 1.0)
    h2 = jnp.dot(hid, w2_ref[...], preferred_element_type=jnp.float32)
    als2 = jnp.sum(h2 * as2_ref[...], axis=1, keepdims=True)
    ald2 = jnp.sum(h2 * ad2_ref[...], axis=1, keepdims=True)
    pad = jnp.zeros((h2.shape[0], 7), jnp.float32)
    t2_ref[...] = jnp.concatenate([h2, als2, pad], axis=1)
    a2_ref[...] = jnp.broadcast_to(ald2, (h2.shape[0], 8))


def _tc3_body(acc_ref, b2_ref, out_ref):
    a0 = acc_ref[0]
    a1 = acc_ref[1]
    num = a0[:, :40] + a1[:, :40]
    den = a0[:, 40:41] + a1[:, 40:41]
    o = num / (den + 1e-16) + b2_ref[...]
    m = jnp.max(o, axis=1, keepdims=True)
    s = jnp.sum(jnp.exp(o - m), axis=1, keepdims=True)
    out_ref[...] = o - m - jnp.log(s)


def _iota16():
    return lax.broadcasted_iota(jnp.int32, (16,), 0)


def _perm16(x, idx):
    return lax.gather(
        x, idx[:, None],
        lax.GatherDimensionNumbers(offset_dims=(), collapsed_slice_dims=(0,),
                                   start_index_map=(0,)),
        (1,), mode=lax.GatherScatterMode.PROMISE_IN_BOUNDS)


_mesh = plsc.VectorSubcoreMesh(core_axis_name="c", subcore_axis_name="s",
                               num_cores=2, num_subcores=16)


NBUF = 4

_scr1 = ([pltpu.VMEM_SHARED((NP, ROWS1), jnp.float32),
          pltpu.VMEM((EPT,), jnp.int32),
          pltpu.VMEM((EPT,), jnp.int32)]
         + [pltpu.VMEM((C,), jnp.int32) for _ in range(NBUF)]
         + [pltpu.VMEM((C, ROWS1), jnp.float32) for _ in range(NBUF)]
         + [pltpu.VMEM((C, 8), jnp.float32) for _ in range(NBUF)]
         + [pltpu.VMEM((C, ROWS1), jnp.float32) for _ in range(NBUF)]
         + [pltpu.VMEM((C, 16), jnp.float32)]
         + [pltpu.SemaphoreType.DMA for _ in range(2 * NBUF)])


@functools.partial(
    pl.kernel,
    out_type=jax.ShapeDtypeStruct((2, NP, ROWS1), jnp.float32),
    mesh=_mesh,
    compiler_params=pltpu.CompilerParams(use_tc_tiling_on_sc=False,
                                         needs_layout_passes=False),
    scratch_types=_scr1,
)
def _sc1(src_ref, dst_ref, t1_ref, a1_ref, z_ref, out_ref,
         acc, src_v, dst_v, *scr):
    dstc = scr[0:NBUF]
    rows = scr[NBUF:2 * NBUF]
    ad = scr[2 * NBUF:3 * NBUF]
    out = scr[3 * NBUF:4 * NBUF]
    w2_v = scr[4 * NBUF]
    gs = scr[4 * NBUF + 1:4 * NBUF + 1 + NBUF]
    ss = scr[4 * NBUF + 1 + NBUF:4 * NBUF + 1 + 2 * NBUF]
    cid = lax.axis_index("c")
    sid = lax.axis_index("s")
    wid = cid * 16 + sid

    pltpu.sync_copy(z_ref, acc.at[pl.ds(sid * RPT, RPT)])

    pltpu.sync_copy(src_ref.at[pl.ds(wid * EPT, EPT)], src_v)
    pltpu.sync_copy(dst_ref.at[pl.ds(wid * EPT, EPT)], dst_v)
    plsc.subcore_barrier()

    def g_start(cc, b):
        off = cc * C
        pltpu.async_copy(t1_ref.at[src_v.at[pl.ds(off, C)]], rows[b], gs[b])
        pltpu.async_copy(a1_ref.at[dst_v.at[pl.ds(off, C)]], ad[b], gs[b])

    def g_wait(b):
        pltpu.make_async_copy(t1_ref.at[pl.ds(0, C)], rows[b], gs[b]).wait()
        pltpu.make_async_copy(a1_ref.at[pl.ds(0, C)], ad[b], gs[b]).wait()

    def s_start(b):
        pltpu.async_copy(out[b], acc.at[dstc[b]], ss[b], add=True)

    def s_wait(b):
        pltpu.make_async_copy(out[b], acc.at[dstc[b]], ss[b]).wait()

    def compute(cc, b):
        lanes = _iota16()
        off = cc * C
        for j in range(C // 16):
            dstc[b][pl.ds(j * 16, 16)] = dst_v[pl.ds(off + j * 16, 16)]
        for g in range(C // 16):
            er = g * 16 + lanes
            als = [plsc.load_gather(rows[b], [er, jnp.full((16,), 64 + hh, jnp.int32)])
                   for hh in range(8)]
            ald = [plsc.load_gather(ad[b], [er, jnp.full((16,), hh, jnp.int32)])
                   for hh in range(8)]
            ws = []
            for hh in range(8):
                t = als[hh] + ald[hh]
                ws.append(jnp.exp(jnp.maximum(t, 0.2 * t)))
            for hh in range(8):
                ch = jnp.full((16,), hh, jnp.int32)
                plsc.store_scatter(out[b], [er, 64 + ch], ws[hh])
                plsc.store_scatter(w2_v, [er, ch], ws[hh])
                plsc.store_scatter(w2_v, [er, 8 + ch], ws[hh])
        for e in range(C):
            wb = w2_v[e, pl.ds(0, 16)]
            for kk in range(4):
                hv = rows[b][e, pl.ds(kk * 16, 16)]
                out[b][e, pl.ds(kk * 16, 16)] = hv * wb

    for b in range(NBUF):
        g_start(b, b)

    def step(kq, _):
        for b in range(NBUF):
            cc = NBUF * kq + b

            @pl.when(cc < NCH)
            def _():
                g_wait(b)

                @pl.when(cc >= NBUF)
                def _():
                    s_wait(b)

                compute(cc, b)
                s_start(b)

                @pl.when(cc + NBUF < NCH)
                def _():
                    g_start(cc + NBUF, b)

        return 0

    lax.fori_loop(0, (NCH + NBUF - 1) // NBUF, step, 0)
    for b in range(NBUF):
        s_wait(b)
    plsc.subcore_barrier()
    base = sid * RPT
    pltpu.sync_copy(acc.at[pl.ds(base, RPT)],
                    out_ref.at[cid, pl.ds(base, RPT)])


_scr2 = ([pltpu.VMEM_SHARED((NP, ROWS2), jnp.float32),
          pltpu.VMEM((EPT,), jnp.int32),
          pltpu.VMEM((EPT,), jnp.int32),
          pltpu.VMEM((N,), jnp.float32)]
         + [pltpu.VMEM((C,), jnp.int32) for _ in range(NBUF)]
         + [pltpu.VMEM((C, ROWS2), jnp.float32) for _ in range(NBUF)]
         + [pltpu.VMEM((C, ROWS2), jnp.float32) for _ in range(NBUF)]
         + [pltpu.SemaphoreType.DMA for _ in range(2 * NBUF)])


@functools.partial(
    pl.kernel,
    out_type=jax.ShapeDtypeStruct((2, NP, ROWS2), jnp.float32),
    mesh=_mesh,
    compiler_params=pltpu.CompilerParams(use_tc_tiling_on_sc=False,
                                         needs_layout_passes=False),
    scratch_types=_scr2,
)
def _sc2(src_ref, dst_ref, t2_ref, a2_ref, z_ref, out_ref,
         acc, src_v, dst_v, a2l, *scr):
    dstc = scr[0:NBUF]
    rows = scr[NBUF:2 * NBUF]
    out = scr[2 * NBUF:3 * NBUF]
    gs = scr[3 * NBUF:3 * NBUF + NBUF]
    ss = scr[3 * NBUF + NBUF:3 * NBUF + 2 * NBUF]
    cid = lax.axis_index("c")
    sid = lax.axis_index("s")
    wid = cid * 16 + sid

    pltpu.sync_copy(z_ref, acc.at[pl.ds(sid * RPT, RPT)])
    pltpu.sync_copy(src_ref.at[pl.ds(wid * EPT, EPT)], src_v)
    pltpu.sync_copy(dst_ref.at[pl.ds(wid * EPT, EPT)], dst_v)
    pltpu.sync_copy(a2_ref, a2l)
    plsc.subcore_barrier()

    def g_start(cc, b):
        off = cc * C
        pltpu.async_copy(t2_ref.at[src_v.at[pl.ds(off, C)]], rows[b], gs[b])

    def g_wait(b):
        pltpu.make_async_copy(t2_ref.at[pl.ds(0, C)], rows[b], gs[b]).wait()

    def s_start(b):
        pltpu.async_copy(out[b], acc.at[dstc[b]], ss[b], add=True)

    def s_wait(b):
        pltpu.make_async_copy(out[b], acc.at[dstc[b]], ss[b]).wait()

    def compute(cc, b):
        lanes = _iota16()
        c40 = jnp.full((16,), 40, jnp.int32)
        off = cc * C
        dst16 = []
        for j in range(C // 16):
            d = dst_v[pl.ds(off + j * 16, 16)]
            dst16.append(d)
            dstc[b][pl.ds(j * 16, 16)] = d
        ers = [g * 16 + lanes for g in range(C // 16)]
        als = [plsc.load_gather(rows[b], [er, c40]) for er in ers]
        ald = [plsc.load_gather(a2l, [dst16[g]]) for g in range(C // 16)]
        ws = []
        for g in range(C // 16):
            t = als[g] + ald[g]
            ws.append(jnp.exp(jnp.maximum(t, 0.2 * t)))
        for e in range(C):
            wb = _perm16(ws[e // 16], jnp.full((16,), e % 16, jnp.int32))
            for kk in range(3):
                hv = rows[b][e, pl.ds(kk * 16, 16)]
                out[b][e, pl.ds(kk * 16, 16)] = hv * wb
        for g in range(C // 16):
            plsc.store_scatter(out[b], [ers[g], c40], ws[g])

    for b in range(NBUF):
        g_start(b, b)

    def step(kq, _):
        for b in range(NBUF):
            cc = NBUF * kq + b

            @pl.when(cc < NCH)
            def _():
                g_wait(b)

                @pl.when(cc >= NBUF)
                def _():
                    s_wait(b)

                compute(cc, b)
                s_start(b)

                @pl.when(cc + NBUF < NCH)
                def _():
                    g_start(cc + NBUF, b)

        return 0

    lax.fori_loop(0, (NCH + NBUF - 1) // NBUF, step, 0)
    for b in range(NBUF):
        s_wait(b)
    plsc.subcore_barrier()
    base = sid * RPT
    pltpu.sync_copy(acc.at[pl.ds(base, RPT)],
                    out_ref.at[cid, pl.ds(base, RPT)])


def kernel(x, edge_index, W1, a_src1, a_dst1, b1, W2, a_src2, a_dst2, b2):
    f32 = jnp.float32
    c = jnp.arange(64)
    P64 = (c[:, None] == ((c % 8) * 8 + c // 8)[None, :]).astype(f32)
    W1t = W1 @ P64
    selt = (jnp.arange(64)[:, None] % 8) == jnp.arange(8)[None, :]
    As = jnp.where(selt, jnp.take(a_src1.reshape(64), (c % 8) * 8 + c // 8)[:, None], 0.0).astype(f32)
    Ad = jnp.where(selt, jnp.take(a_dst1.reshape(64), (c % 8) * 8 + c // 8)[:, None], 0.0).astype(f32)
    E8 = selt.astype(f32).T
    b1t = jnp.take(b1, (c % 8) * 8 + c // 8)
    W2e = P64.T @ W2

    T1, A1 = pl.pallas_call(
        _tc1_body,
        grid=(N // _BN,),
        in_specs=[
            pl.BlockSpec((_BN, NFEAT), lambda i: (i, 0)),
            pl.BlockSpec((NFEAT, 64), lambda i: (0, 0)),
            pl.BlockSpec((64, 8), lambda i: (0, 0)),
            pl.BlockSpec((64, 8), lambda i: (0, 0)),
        ],
        out_specs=[
            pl.BlockSpec((_BN, ROWS1), lambda i: (i, 0)),
            pl.BlockSpec((_BN, 8), lambda i: (i, 0)),
        ],
        out_shape=[
            jax.ShapeDtypeStruct((N, ROWS1), f32),
            jax.ShapeDtypeStruct((N, 8), f32),
        ],
    )(x, W1t, As, Ad)

    src = edge_index[0]
    dst = edge_index[1]
    z1 = jnp.zeros((RPT, ROWS1), f32)
    acc1 = _sc1(src, dst, T1, A1, z1)

    T2, A2 = pl.pallas_call(
        _tc2_body,
        grid=(N // _BN,),
        in_specs=[
            pl.BlockSpec((2, _BN, ROWS1), lambda i: (0, i, 0)),
            pl.BlockSpec((1, 64), lambda i: (0, 0)),
            pl.BlockSpec((64, NCLASS), lambda i: (0, 0)),
            pl.BlockSpec((1, NCLASS), lambda i: (0, 0)),
            pl.BlockSpec((1, NCLASS), lambda i: (0, 0)),
            pl.BlockSpec((8, 64), lambda i: (0, 0)),
        ],
        out_specs=[
            pl.BlockSpec((_BN, ROWS2), lambda i: (i, 0)),
            pl.BlockSpec((_BN, 8), lambda i: (i, 0)),
        ],
        out_shape=[
            jax.ShapeDtypeStruct((N, ROWS2), f32),
            jax.ShapeDtypeStruct((N, 8), f32),
        ],
    )(acc1, b1t.reshape(1, 64), W2e, a_src2, a_dst2, E8)

    z2 = jnp.zeros((RPT, ROWS2), f32)
    acc2 = _sc2(src, dst, T2, A2[:, 0], z2)

    out = pl.pallas_call(
        _tc3_body,
        grid=(N // _BN,),
        in_specs=[
            pl.BlockSpec((2, _BN, ROWS2), lambda i: (0, i, 0)),
            pl.BlockSpec((1, NCLASS), lambda i: (0, 0)),
        ],
        out_specs=pl.BlockSpec((_BN, NCLASS), lambda i: (i, 0)),
        out_shape=jax.ShapeDtypeStruct((N, NCLASS), f32),
    )(acc2, b2.reshape(1, NCLASS))

    return out


# speedup vs baseline: 1.0471x; 1.0471x over previous
import functools

import jax
import jax.numpy as jnp
from jax import lax
from jax.experimental import pallas as pl
from jax.experimental.pallas import tpu as pltpu
from jax.experimental.pallas import tpu_sc as plsc

N = 10000
E = 320000
NFEAT = 128
NHID = 8
NHEADS = 8
NCLASS = 40

NTILE = 32
EPT = E // NTILE
C = 80
NCH = EPT // C
ROWS1 = 72
ROWS2 = 48
NP = 10240
RPT = NP // 16

_BN = 1000


def _tc1_body(x_ref, w1_ref, as_ref, ad_ref, t1_ref, a1_ref):
    h = jnp.dot(x_ref[...], w1_ref[...], preferred_element_type=jnp.float32)
    als = jnp.dot(h, as_ref[...], preferred_element_type=jnp.float32)
    ald = jnp.dot(h, ad_ref[...], preferred_element_type=jnp.float32)
    t1_ref[...] = jnp.concatenate([h, als], axis=1)
    a1_ref[...] = ald


def _tc2_body(acc_ref, b1_ref, w2_ref, as2_ref, ad2_ref, e8_ref, t2_ref, a2_ref):
    a0 = acc_ref[0]
    a1 = acc_ref[1]
    num = a0[:, :64] + a1[:, :64]
    den = a0[:, 64:] + a1[:, 64:]
    r = 1.0 / (den + 1e-16)
    rexp = jnp.dot(r, e8_ref[...], preferred_element_type=jnp.float32)
    hid = num * rexp + b1_ref[...]
    hid = jnp.where(hid > 0, hid, jnp.exp(hid) - 1.0)
    h2 = jnp.dot(hid, w2_ref[...], preferred_element_type=jnp.float32)
    als2 = jnp.sum(h2 * as2_ref[...], axis=1, keepdims=True)
    ald2 = jnp.sum(h2 * ad2_ref[...], axis=1, keepdims=True)
    pad = jnp.zeros((h2.shape[0], 7), jnp.float32)
    t2_ref[...] = jnp.concatenate([h2, als2, pad], axis=1)
    a2_ref[...] = jnp.broadcast_to(ald2, (h2.shape[0], 8))


def _tc3_body(acc_ref, b2_ref, out_ref):
    a0 = acc_ref[0]
    a1 = acc_ref[1]
    num = a0[:, :40] + a1[:, :40]
    den = a0[:, 40:41] + a1[:, 40:41]
    o = num / (den + 1e-16) + b2_ref[...]
    m = jnp.max(o, axis=1, keepdims=True)
    s = jnp.sum(jnp.exp(o - m), axis=1, keepdims=True)
    out_ref[...] = o - m - jnp.log(s)


def _iota16():
    return lax.broadcasted_iota(jnp.int32, (16,), 0)


def _perm16(x, idx):
    return lax.gather(
        x, idx[:, None],
        lax.GatherDimensionNumbers(offset_dims=(), collapsed_slice_dims=(0,),
                                   start_index_map=(0,)),
        (1,), mode=lax.GatherScatterMode.PROMISE_IN_BOUNDS)


_mesh = plsc.VectorSubcoreMesh(core_axis_name="c", subcore_axis_name="s",
                               num_cores=2, num_subcores=16)


NBUF = 2

_scr1 = ([pltpu.VMEM_SHARED((NP, ROWS1), jnp.float32),
          pltpu.VMEM((EPT,), jnp.int32),
          pltpu.VMEM((EPT,), jnp.int32)]
         + [pltpu.VMEM((C,), jnp.int32) for _ in range(NBUF)]
         + [pltpu.VMEM((C, ROWS1), jnp.float32) for _ in range(NBUF)]
         + [pltpu.VMEM((C, 8), jnp.float32) for _ in range(NBUF)]
         + [pltpu.VMEM((C, ROWS1), jnp.float32) for _ in range(NBUF)]
         + [pltpu.VMEM((C, 16), jnp.float32)]
         + [pltpu.SemaphoreType.DMA for _ in range(2 * NBUF)])


@functools.partial(
    pl.kernel,
    out_type=jax.ShapeDtypeStruct((2, NP, ROWS1), jnp.float32),
    mesh=_mesh,
    compiler_params=pltpu.CompilerParams(use_tc_tiling_on_sc=False,
                                         needs_layout_passes=False),
    scratch_types=_scr1,
)
def _sc1(src_ref, dst_ref, t1_ref, a1_ref, z_ref, out_ref,
         acc, src_v, dst_v, *scr):
    dstc = scr[0:NBUF]
    rows = scr[NBUF:2 * NBUF]
    ad = scr[2 * NBUF:3 * NBUF]
    out = scr[3 * NBUF:4 * NBUF]
    w2_v = scr[4 * NBUF]
    gs = scr[4 * NBUF + 1:4 * NBUF + 1 + NBUF]
    ss = scr[4 * NBUF + 1 + NBUF:4 * NBUF + 1 + 2 * NBUF]
    cid = lax.axis_index("c")
    sid = lax.axis_index("s")
    wid = cid * 16 + sid

    pltpu.sync_copy(z_ref, acc.at[pl.ds(sid * RPT, RPT)])

    pltpu.sync_copy(src_ref.at[pl.ds(wid * EPT, EPT)], src_v)
    pltpu.sync_copy(dst_ref.at[pl.ds(wid * EPT, EPT)], dst_v)
    plsc.subcore_barrier()

    def g_start(cc, b):
        off = cc * C
        pltpu.async_copy(t1_ref.at[src_v.at[pl.ds(off, C)]], rows[b], gs[b])
        pltpu.async_copy(a1_ref.at[dst_v.at[pl.ds(off, C)]], ad[b], gs[b])

    def g_wait(b):
        pltpu.make_async_copy(t1_ref.at[pl.ds(0, C)], rows[b], gs[b]).wait()
        pltpu.make_async_copy(a1_ref.at[pl.ds(0, C)], ad[b], gs[b]).wait()

    def s_start(b):
        pltpu.async_copy(out[b], acc.at[dstc[b]], ss[b], add=True)

    def s_wait(b):
        pltpu.make_async_copy(out[b], acc.at[dstc[b]], ss[b]).wait()

    def compute(cc, b):
        lanes = _iota16()
        off = cc * C
        for j in range(C // 16):
            dstc[b][pl.ds(j * 16, 16)] = dst_v[pl.ds(off + j * 16, 16)]
        for g in range(C // 16):
            er = g * 16 + lanes
            als = [plsc.load_gather(rows[b], [er, jnp.full((16,), 64 + hh, jnp.int32)])
                   for hh in range(8)]
            ald = [plsc.load_gather(ad[b], [er, jnp.full((16,), hh, jnp.int32)])
                   for hh in range(8)]
            ws = []
            for hh in range(8):
                t = als[hh] + ald[hh]
                ws.append(jnp.exp(jnp.maximum(t, 0.2 * t)))
            for hh in range(8):
                ch = jnp.full((16,), hh, jnp.int32)
                plsc.store_scatter(out[b], [er, 64 + ch], ws[hh])
                plsc.store_scatter(w2_v, [er, ch], ws[hh])
                plsc.store_scatter(w2_v, [er, 8 + ch], ws[hh])
        for e in range(C):
            wb = w2_v[e, pl.ds(0, 16)]
            for kk in range(4):
                hv = rows[b][e, pl.ds(kk * 16, 16)]
                out[b][e, pl.ds(kk * 16, 16)] = hv * wb

    for b in range(NBUF):
        g_start(b, b)

    def step(kq, _):
        for b in range(NBUF):
            cc = NBUF * kq + b

            @pl.when(cc < NCH)
            def _():
                g_wait(b)

                @pl.when(cc >= NBUF)
                def _():
                    s_wait(b)

                compute(cc, b)
                s_start(b)

                @pl.when(cc + NBUF < NCH)
                def _():
                    g_start(cc + NBUF, b)

        return 0

    lax.fori_loop(0, (NCH + NBUF - 1) // NBUF, step, 0)
    for b in range(NBUF):
        s_wait(b)
    plsc.subcore_barrier()
    base = sid * RPT
    pltpu.sync_copy(acc.at[pl.ds(base, RPT)],
                    out_ref.at[cid, pl.ds(base, RPT)])


_scr2 = ([pltpu.VMEM_SHARED((NP, ROWS2), jnp.float32),
          pltpu.VMEM((EPT,), jnp.int32),
          pltpu.VMEM((EPT,), jnp.int32),
          pltpu.VMEM((N,), jnp.float32)]
         + [pltpu.VMEM((C,), jnp.int32) for _ in range(NBUF)]
         + [pltpu.VMEM((C, ROWS2), jnp.float32) for _ in range(NBUF)]
         + [pltpu.VMEM((C, ROWS2), jnp.float32) for _ in range(NBUF)]
         + [pltpu.SemaphoreType.DMA for _ in range(2 * NBUF)])


@functools.partial(
    pl.kernel,
    out_type=jax.ShapeDtypeStruct((2, NP, ROWS2), jnp.float32),
    mesh=_mesh,
    compiler_params=pltpu.CompilerParams(use_tc_tiling_on_sc=False,
                                         needs_layout_passes=False),
    scratch_types=_scr2,
)
def _sc2(src_ref, dst_ref, t2_ref, a2_ref, z_ref, out_ref,
         acc, src_v, dst_v, a2l, *scr):
    dstc = scr[0:NBUF]
    rows = scr[NBUF:2 * NBUF]
    out = scr[2 * NBUF:3 * NBUF]
    gs = scr[3 * NBUF:3 * NBUF + NBUF]
    ss = scr[3 * NBUF + NBUF:3 * NBUF + 2 * NBUF]
    cid = lax.axis_index("c")
    sid = lax.axis_index("s")
    wid = cid * 16 + sid

    pltpu.sync_copy(z_ref, acc.at[pl.ds(sid * RPT, RPT)])
    pltpu.sync_copy(src_ref.at[pl.ds(wid * EPT, EPT)], src_v)
    pltpu.sync_copy(dst_ref.at[pl.ds(wid * EPT, EPT)], dst_v)
    pltpu.sync_copy(a2_ref, a2l)
    plsc.subcore_barrier()

    def g_start(cc, b):
        off = cc * C
        pltpu.async_copy(t2_ref.at[src_v.at[pl.ds(off, C)]], rows[b], gs[b])

    def g_wait(b):
        pltpu.make_async_copy(t2_ref.at[pl.ds(0, C)], rows[b], gs[b]).wait()

    def s_start(b):
        pltpu.async_copy(out[b], acc.at[dstc[b]], ss[b], add=True)

    def s_wait(b):
        pltpu.make_async_copy(out[b], acc.at[dstc[b]], ss[b]).wait()

    def compute(cc, b):
        lanes = _iota16()
        c40 = jnp.full((16,), 40, jnp.int32)
        off = cc * C
        dst16 = []
        for j in range(C // 16):
            d = dst_v[pl.ds(off + j * 16, 16)]
            dst16.append(d)
            dstc[b][pl.ds(j * 16, 16)] = d
        ers = [g * 16 + lanes for g in range(C // 16)]
        als = [plsc.load_gather(rows[b], [er, c40]) for er in ers]
        ald = [plsc.load_gather(a2l, [dst16[g]]) for g in range(C // 16)]
        ws = []
        for g in range(C // 16):
            t = als[g] + ald[g]
            ws.append(jnp.exp(jnp.maximum(t, 0.2 * t)))
        for e in range(C):
            wb = _perm16(ws[e // 16], jnp.full((16,), e % 16, jnp.int32))
            for kk in range(3):
                hv = rows[b][e, pl.ds(kk * 16, 16)]
                out[b][e, pl.ds(kk * 16, 16)] = hv * wb
        for g in range(C // 16):
            plsc.store_scatter(out[b], [ers[g], c40], ws[g])

    for b in range(NBUF):
        g_start(b, b)

    def step(kq, _):
        for b in range(NBUF):
            cc = NBUF * kq + b

            @pl.when(cc < NCH)
            def _():
                g_wait(b)

                @pl.when(cc >= NBUF)
                def _():
                    s_wait(b)

                compute(cc, b)
                s_start(b)

                @pl.when(cc + NBUF < NCH)
                def _():
                    g_start(cc + NBUF, b)

        return 0

    lax.fori_loop(0, (NCH + NBUF - 1) // NBUF, step, 0)
    for b in range(NBUF):
        s_wait(b)
    plsc.subcore_barrier()
    base = sid * RPT
    pltpu.sync_copy(acc.at[pl.ds(base, RPT)],
                    out_ref.at[cid, pl.ds(base, RPT)])


def kernel(x, edge_index, W1, a_src1, a_dst1, b1, W2, a_src2, a_dst2, b2):
    f32 = jnp.float32
    c = jnp.arange(64)
    P64 = (c[:, None] == ((c % 8) * 8 + c // 8)[None, :]).astype(f32)
    W1t = W1 @ P64
    selt = (jnp.arange(64)[:, None] % 8) == jnp.arange(8)[None, :]
    As = jnp.where(selt, jnp.take(a_src1.reshape(64), (c % 8) * 8 + c // 8)[:, None], 0.0).astype(f32)
    Ad = jnp.where(selt, jnp.take(a_dst1.reshape(64), (c % 8) * 8 + c // 8)[:, None], 0.0).astype(f32)
    E8 = selt.astype(f32).T
    b1t = jnp.take(b1, (c % 8) * 8 + c // 8)
    W2e = P64.T @ W2

    T1, A1 = pl.pallas_call(
        _tc1_body,
        grid=(N // _BN,),
        in_specs=[
            pl.BlockSpec((_BN, NFEAT), lambda i: (i, 0)),
            pl.BlockSpec((NFEAT, 64), lambda i: (0, 0)),
            pl.BlockSpec((64, 8), lambda i: (0, 0)),
            pl.BlockSpec((64, 8), lambda i: (0, 0)),
        ],
        out_specs=[
            pl.BlockSpec((_BN, ROWS1), lambda i: (i, 0)),
            pl.BlockSpec((_BN, 8), lambda i: (i, 0)),
        ],
        out_shape=[
            jax.ShapeDtypeStruct((N, ROWS1), f32),
            jax.ShapeDtypeStruct((N, 8), f32),
        ],
    )(x, W1t, As, Ad)

    src = edge_index[0]
    dst = edge_index[1]
    z1 = jnp.zeros((RPT, ROWS1), f32)
    acc1 = _sc1(src, dst, T1, A1, z1)

    T2, A2 = pl.pallas_call(
        _tc2_body,
        grid=(N // _BN,),
        in_specs=[
            pl.BlockSpec((2, _BN, ROWS1), lambda i: (0, i, 0)),
            pl.BlockSpec((1, 64), lambda i: (0, 0)),
            pl.BlockSpec((64, NCLASS), lambda i: (0, 0)),
            pl.BlockSpec((1, NCLASS), lambda i: (0, 0)),
            pl.BlockSpec((1, NCLASS), lambda i: (0, 0)),
            pl.BlockSpec((8, 64), lambda i: (0, 0)),
        ],
        out_specs=[
            pl.BlockSpec((_BN, ROWS2), lambda i: (i, 0)),
            pl.BlockSpec((_BN, 8), lambda i: (i, 0)),
        ],
        out_shape=[
            jax.ShapeDtypeStruct((N, ROWS2), f32),
            jax.ShapeDtypeStruct((N, 8), f32),
        ],
    )(acc1, b1t.reshape(1, 64), W2e, a_src2, a_dst2, E8)

    z2 = jnp.zeros((RPT, ROWS2), f32)
    acc2 = _sc2(src, dst, T2, A2[:, 0], z2)

    out = pl.pallas_call(
        _tc3_body,
        grid=(N // _BN,),
        in_specs=[
            pl.BlockSpec((2, _BN, ROWS2), lambda i: (0, i, 0)),
            pl.BlockSpec((1, NCLASS), lambda i: (0, 0)),
        ],
        out_specs=pl.BlockSpec((_BN, NCLASS), lambda i: (i, 0)),
        out_shape=jax.ShapeDtypeStruct((N, NCLASS), f32),
    )(acc2, b2.reshape(1, NCLASS))

    return out
